# Initial kernel scaffold; baseline (speedup 1.0000x reference)
import jax
import jax.numpy as jnp
from jax.experimental import pallas as pl


def kernel(input_ids, table, W_c, b_c, W_g, b_g):
    raise NotImplementedError("write your pallas kernel here")


# speedup vs baseline: 2.6430x; 2.6430x over previous
import functools

import jax
import jax.numpy as jnp
from jax import lax
from jax.experimental import pallas as pl
from jax.experimental.pallas import tpu as pltpu
from jax.experimental.pallas import tpu_sc as plsc

VOCAB = 100000
EMBED = 128
ROW_BLOCK = 2000


def _transform_body(emb_ref, wc_ref, bc_ref, wg_ref, bg_ref, out_ref):
    emb = emb_ref[...]
    dims = (((1,), (1,)), ((), ()))
    ctx = lax.dot_general(emb, wc_ref[...], dims,
                          preferred_element_type=jnp.float32) + bc_ref[...]
    gate_lin = lax.dot_general(emb, wg_ref[...], dims,
                               preferred_element_type=jnp.float32) + bg_ref[...]
    out_ref[...] = emb + jax.nn.sigmoid(gate_lin) * ctx


def _transform_table(table, W_c, b_c, W_g, b_g):
    n_blocks = VOCAB // ROW_BLOCK
    return pl.pallas_call(
        _transform_body,
        grid=(n_blocks,),
        in_specs=[
            pl.BlockSpec((ROW_BLOCK, EMBED), lambda i: (i, 0)),
            pl.BlockSpec((EMBED, EMBED), lambda i: (0, 0)),
            pl.BlockSpec((1, EMBED), lambda i: (0, 0)),
            pl.BlockSpec((EMBED, EMBED), lambda i: (0, 0)),
            pl.BlockSpec((1, EMBED), lambda i: (0, 0)),
        ],
        out_specs=pl.BlockSpec((ROW_BLOCK, EMBED), lambda i: (i, 0)),
        out_shape=jax.ShapeDtypeStruct((VOCAB, EMBED), jnp.float32),
    )(table, W_c, b_c.reshape(1, EMBED), W_g, b_g.reshape(1, EMBED))


_NW = 32
_CH = 128


def _make_sc_gather(n_tokens):
    b_per_w = n_tokens // _NW
    n_chunks = b_per_w // _CH
    mesh = plsc.VectorSubcoreMesh(core_axis_name="c", subcore_axis_name="s")

    @functools.partial(
        pl.kernel,
        mesh=mesh,
        out_type=jax.ShapeDtypeStruct((n_tokens, EMBED), jnp.float32),
        scratch_types=[
            pltpu.VMEM((_CH,), jnp.int32),
            pltpu.VMEM((_CH, EMBED), jnp.float32),
            pltpu.SemaphoreType.DMA,
        ],
    )
    def gather_kernel(o_hbm, idx_hbm, out_hbm, idx_v, rows_v, sem):
        wid = lax.axis_index("s") * 2 + lax.axis_index("c")
        base = wid * b_per_w

        def body(i, carry):
            off = base + i * _CH
            pltpu.sync_copy(idx_hbm.at[pl.ds(off, _CH)], idx_v)
            pltpu.async_copy(o_hbm.at[idx_v], rows_v, sem).wait()
            pltpu.sync_copy(rows_v, out_hbm.at[pl.ds(off, _CH)])
            return carry

        lax.fori_loop(0, n_chunks, body, 0)

    return gather_kernel


def kernel(input_ids, table, W_c, b_c, W_g, b_g):
    transformed = _transform_table(table, W_c, b_c, W_g, b_g)
    flat_ids = input_ids.reshape(-1).astype(jnp.int32)
    out = _make_sc_gather(flat_ids.shape[0])(transformed, flat_ids)
    return out.reshape(input_ids.shape + (EMBED,))


# speedup vs baseline: 5.1394x; 1.9446x over previous
import functools

import jax
import jax.numpy as jnp
from jax import lax
from jax.experimental import pallas as pl
from jax.experimental.pallas import tpu as pltpu
from jax.experimental.pallas import tpu_sc as plsc

VOCAB = 100000
EMBED = 128
ROW_BLOCK = 2000


def _transform_body(emb_ref, wc_ref, bc_ref, wg_ref, bg_ref, out_ref):
    emb = emb_ref[...]
    dims = (((1,), (1,)), ((), ()))
    ctx = lax.dot_general(emb, wc_ref[...], dims,
                          preferred_element_type=jnp.float32) + bc_ref[...]
    gate_lin = lax.dot_general(emb, wg_ref[...], dims,
                               preferred_element_type=jnp.float32) + bg_ref[...]
    out_ref[...] = emb + jax.nn.sigmoid(gate_lin) * ctx


def _transform_table(table, W_c, b_c, W_g, b_g):
    n_blocks = VOCAB // ROW_BLOCK
    return pl.pallas_call(
        _transform_body,
        grid=(n_blocks,),
        in_specs=[
            pl.BlockSpec((ROW_BLOCK, EMBED), lambda i: (i, 0)),
            pl.BlockSpec((EMBED, EMBED), lambda i: (0, 0)),
            pl.BlockSpec((1, EMBED), lambda i: (0, 0)),
            pl.BlockSpec((EMBED, EMBED), lambda i: (0, 0)),
            pl.BlockSpec((1, EMBED), lambda i: (0, 0)),
        ],
        out_specs=pl.BlockSpec((ROW_BLOCK, EMBED), lambda i: (i, 0)),
        out_shape=jax.ShapeDtypeStruct((VOCAB, EMBED), jnp.float32),
    )(table, W_c, b_c.reshape(1, EMBED), W_g, b_g.reshape(1, EMBED))


_NW = 32
_CH = 128


def _make_sc_gather(n_tokens):
    b_per_w = n_tokens // _NW
    n_chunks = b_per_w // _CH
    mesh = plsc.VectorSubcoreMesh(core_axis_name="c", subcore_axis_name="s")

    @functools.partial(
        pl.kernel,
        mesh=mesh,
        out_type=jax.ShapeDtypeStruct((n_tokens, EMBED), jnp.float32),
        scratch_types=[
            pltpu.VMEM((_CH,), jnp.int32),
            pltpu.VMEM((_CH, EMBED), jnp.float32),
            pltpu.SemaphoreType.DMA,
        ],
    )
    def gather_kernel(o_hbm, idx_hbm, out_hbm, idx_v, rows_v, sem):
        wid = lax.axis_index("s") * 2 + lax.axis_index("c")
        base = wid * b_per_w

        def body(i, carry):
            off = base + i * _CH
            pltpu.sync_copy(idx_hbm.at[pl.ds(off, _CH)], idx_v)
            pltpu.async_copy(o_hbm.at[idx_v], rows_v, sem).wait()
            pltpu.sync_copy(rows_v, out_hbm.at[pl.ds(off, _CH)])
            return carry

        lax.fori_loop(0, n_chunks, body, 0)

    return gather_kernel


def kernel(input_ids, table, W_c, b_c, W_g, b_g):
    transformed = _transform_table(table, W_c, b_c, W_g, b_g)
    B, L = input_ids.shape
    flat_ids = input_ids.T.reshape(-1).astype(jnp.int32)
    out = _make_sc_gather(flat_ids.shape[0])(transformed, flat_ids)
    return out.reshape(L, B, EMBED).transpose(1, 0, 2)


# speedup vs baseline: 7.2409x; 1.4089x over previous
import functools

import jax
import jax.numpy as jnp
from jax import lax
from jax.experimental import pallas as pl
from jax.experimental.pallas import tpu as pltpu
from jax.experimental.pallas import tpu_sc as plsc

VOCAB = 100000
EMBED = 128
ROW_BLOCK = 2000


def _transform_body(emb_ref, wc_ref, bc_ref, wg_ref, bg_ref, out_ref):
    emb = emb_ref[...]
    dims = (((1,), (1,)), ((), ()))
    ctx = lax.dot_general(emb, wc_ref[...], dims,
                          preferred_element_type=jnp.float32) + bc_ref[...]
    gate_lin = lax.dot_general(emb, wg_ref[...], dims,
                               preferred_element_type=jnp.float32) + bg_ref[...]
    out_ref[...] = emb + jax.nn.sigmoid(gate_lin) * ctx


def _transform_table(table, W_c, b_c, W_g, b_g):
    n_blocks = VOCAB // ROW_BLOCK
    return pl.pallas_call(
        _transform_body,
        grid=(n_blocks,),
        in_specs=[
            pl.BlockSpec((ROW_BLOCK, EMBED), lambda i: (i, 0)),
            pl.BlockSpec((EMBED, EMBED), lambda i: (0, 0)),
            pl.BlockSpec((1, EMBED), lambda i: (0, 0)),
            pl.BlockSpec((EMBED, EMBED), lambda i: (0, 0)),
            pl.BlockSpec((1, EMBED), lambda i: (0, 0)),
        ],
        out_specs=pl.BlockSpec((ROW_BLOCK, EMBED), lambda i: (i, 0)),
        out_shape=jax.ShapeDtypeStruct((VOCAB, EMBED), jnp.float32),
    )(table, W_c, b_c.reshape(1, EMBED), W_g, b_g.reshape(1, EMBED))


_NW = 32
_CH = 128
_NBUF = 5


def _make_sc_gather(n_tokens):
    b_per_w = n_tokens // _NW
    n_chunks = b_per_w // _CH
    n_outer = n_chunks // _NBUF
    mesh = plsc.VectorSubcoreMesh(core_axis_name="c", subcore_axis_name="s")

    @functools.partial(
        pl.kernel,
        mesh=mesh,
        out_type=jax.ShapeDtypeStruct((n_tokens, EMBED), jnp.float32),
        scratch_types=[
            pltpu.VMEM((n_chunks, _CH), jnp.int32),
            pltpu.VMEM((_NBUF, _CH, EMBED), jnp.float32),
            pltpu.SemaphoreType.DMA((_NBUF,)),
            pltpu.SemaphoreType.DMA((_NBUF,)),
        ],
    )
    def gather_kernel(o_hbm, idx_hbm, out_hbm, idx_v, rows_v, gsem, wsem):
        wid = lax.axis_index("s") * 2 + lax.axis_index("c")
        base = wid * b_per_w
        pltpu.sync_copy(idx_hbm.at[wid], idx_v)

        def start_gather(j, b):
            pltpu.make_async_copy(
                o_hbm.at[idx_v.at[j]], rows_v.at[b], gsem.at[b]).start()

        def wait_gather(j, b):
            pltpu.make_async_copy(
                o_hbm.at[idx_v.at[j]], rows_v.at[b], gsem.at[b]).wait()

        def start_write(j, b):
            pltpu.make_async_copy(
                rows_v.at[b], out_hbm.at[pl.ds(base + j * _CH, _CH)],
                wsem.at[b]).start()

        def wait_write(b):
            pltpu.make_async_copy(
                rows_v.at[b], out_hbm.at[pl.ds(base, _CH)], wsem.at[b]).wait()

        for b in range(_NBUF - 1):
            start_gather(b, b)

        def outer(i, carry):
            for b in range(_NBUF):
                j = i * _NBUF + b
                nb = (b + _NBUF - 1) % _NBUF
                if b == 0:
                    @pl.when(i > 0)
                    def _():
                        wait_write(nb)
                    start_gather(j + _NBUF - 1, nb)
                else:
                    @pl.when(i < n_outer - 1)
                    def _():
                        wait_write(nb)
                        start_gather(j + _NBUF - 1, nb)
                wait_gather(j, b)
                start_write(j, b)
            return carry

        lax.fori_loop(0, n_outer, outer, 0)
        for b in range(_NBUF):
            wait_write(b)

    return gather_kernel


def kernel(input_ids, table, W_c, b_c, W_g, b_g):
    transformed = _transform_table(table, W_c, b_c, W_g, b_g)
    B, L = input_ids.shape
    flat_ids = input_ids.T.reshape(-1).astype(jnp.int32)
    n_tokens = flat_ids.shape[0]
    idx3 = flat_ids.reshape(_NW, n_tokens // (_NW * _CH), _CH)
    out = _make_sc_gather(n_tokens)(transformed, idx3)
    return out.reshape(L, B, EMBED).transpose(1, 0, 2)


# speedup vs baseline: 8.4890x; 1.1724x over previous
import functools

import jax
import jax.numpy as jnp
from jax import lax
from jax.experimental import pallas as pl
from jax.experimental.pallas import tpu as pltpu
from jax.experimental.pallas import tpu_sc as plsc

VOCAB = 100000
EMBED = 128
ROW_BLOCK = 10000


def _transform_body(emb_ref, wc_ref, bc_ref, wg_ref, bg_ref, out_ref):
    emb = emb_ref[...]
    dims = (((1,), (1,)), ((), ()))
    ctx = lax.dot_general(emb, wc_ref[...], dims,
                          preferred_element_type=jnp.float32) + bc_ref[...]
    gate_lin = lax.dot_general(emb, wg_ref[...], dims,
                               preferred_element_type=jnp.float32) + bg_ref[...]
    out_ref[...] = emb + jax.nn.sigmoid(gate_lin) * ctx


def _transform_table(table, W_c, b_c, W_g, b_g):
    n_blocks = VOCAB // ROW_BLOCK
    return pl.pallas_call(
        _transform_body,
        grid=(n_blocks,),
        in_specs=[
            pl.BlockSpec((ROW_BLOCK, EMBED), lambda i: (i, 0)),
            pl.BlockSpec((EMBED, EMBED), lambda i: (0, 0)),
            pl.BlockSpec((1, EMBED), lambda i: (0, 0)),
            pl.BlockSpec((EMBED, EMBED), lambda i: (0, 0)),
            pl.BlockSpec((1, EMBED), lambda i: (0, 0)),
        ],
        out_specs=pl.BlockSpec((ROW_BLOCK, EMBED), lambda i: (i, 0)),
        out_shape=jax.ShapeDtypeStruct((VOCAB, EMBED), jnp.float32),
    )(table, W_c, b_c.reshape(1, EMBED), W_g, b_g.reshape(1, EMBED))


_NW = 32
_CH = 128
_NBUF = 5


def _make_sc_gather(n_tokens):
    b_per_w = n_tokens // _NW
    n_chunks = b_per_w // _CH
    n_outer = n_chunks // _NBUF
    mesh = plsc.VectorSubcoreMesh(core_axis_name="c", subcore_axis_name="s")

    @functools.partial(
        pl.kernel,
        mesh=mesh,
        out_type=jax.ShapeDtypeStruct((n_tokens, EMBED), jnp.float32),
        scratch_types=[
            pltpu.VMEM((n_chunks, _CH), jnp.int32),
            pltpu.VMEM((_NBUF, _CH, EMBED), jnp.float32),
            pltpu.SemaphoreType.DMA((_NBUF,)),
            pltpu.SemaphoreType.DMA((_NBUF,)),
        ],
    )
    def gather_kernel(o_hbm, idx_hbm, out_hbm, idx_v, rows_v, gsem, wsem):
        wid = lax.axis_index("s") * 2 + lax.axis_index("c")
        base = wid * b_per_w
        pltpu.sync_copy(idx_hbm.at[wid], idx_v)

        def start_gather(j, b):
            pltpu.make_async_copy(
                o_hbm.at[idx_v.at[j]], rows_v.at[b], gsem.at[b]).start()

        def wait_gather(j, b):
            pltpu.make_async_copy(
                o_hbm.at[idx_v.at[j]], rows_v.at[b], gsem.at[b]).wait()

        def start_write(j, b):
            pltpu.make_async_copy(
                rows_v.at[b], out_hbm.at[pl.ds(base + j * _CH, _CH)],
                wsem.at[b]).start()

        def wait_write(b):
            pltpu.make_async_copy(
                rows_v.at[b], out_hbm.at[pl.ds(base, _CH)], wsem.at[b]).wait()

        for b in range(_NBUF - 1):
            start_gather(b, b)

        def outer(i, carry):
            for b in range(_NBUF):
                j = i * _NBUF + b
                nb = (b + _NBUF - 1) % _NBUF
                if b == 0:
                    @pl.when(i > 0)
                    def _():
                        wait_write(nb)
                    start_gather(j + _NBUF - 1, nb)
                else:
                    @pl.when(i < n_outer - 1)
                    def _():
                        wait_write(nb)
                        start_gather(j + _NBUF - 1, nb)
                wait_gather(j, b)
                start_write(j, b)
            return carry

        lax.fori_loop(0, n_outer, outer, 0)
        for b in range(_NBUF):
            wait_write(b)

    return gather_kernel


def kernel(input_ids, table, W_c, b_c, W_g, b_g):
    transformed = _transform_table(table, W_c, b_c, W_g, b_g)
    B, L = input_ids.shape
    flat_ids = input_ids.T.reshape(-1).astype(jnp.int32)
    n_tokens = flat_ids.shape[0]
    idx3 = flat_ids.reshape(_NW, n_tokens // (_NW * _CH), _CH)
    out = _make_sc_gather(n_tokens)(transformed, idx3)
    return out.reshape(L, B, EMBED).transpose(1, 0, 2)


# speedup vs baseline: 8.5358x; 1.0055x over previous
import functools

import jax
import jax.numpy as jnp
from jax import lax
from jax.experimental import pallas as pl
from jax.experimental.pallas import tpu as pltpu
from jax.experimental.pallas import tpu_sc as plsc

VOCAB = 100000
EMBED = 128
ROW_BLOCK = 20000


def _transform_body(emb_ref, wc_ref, bc_ref, wg_ref, bg_ref, out_ref):
    emb = emb_ref[...]
    dims = (((1,), (1,)), ((), ()))
    ctx = lax.dot_general(emb, wc_ref[...], dims,
                          preferred_element_type=jnp.float32) + bc_ref[...]
    gate_lin = lax.dot_general(emb, wg_ref[...], dims,
                               preferred_element_type=jnp.float32) + bg_ref[...]
    out_ref[...] = emb + jax.nn.sigmoid(gate_lin) * ctx


def _transform_table(table, W_c, b_c, W_g, b_g):
    n_blocks = VOCAB // ROW_BLOCK
    return pl.pallas_call(
        _transform_body,
        grid=(n_blocks,),
        in_specs=[
            pl.BlockSpec((ROW_BLOCK, EMBED), lambda i: (i, 0)),
            pl.BlockSpec((EMBED, EMBED), lambda i: (0, 0)),
            pl.BlockSpec((1, EMBED), lambda i: (0, 0)),
            pl.BlockSpec((EMBED, EMBED), lambda i: (0, 0)),
            pl.BlockSpec((1, EMBED), lambda i: (0, 0)),
        ],
        out_specs=pl.BlockSpec((ROW_BLOCK, EMBED), lambda i: (i, 0)),
        out_shape=jax.ShapeDtypeStruct((VOCAB, EMBED), jnp.float32),
    )(table, W_c, b_c.reshape(1, EMBED), W_g, b_g.reshape(1, EMBED))


_NW = 32
_CH = 128
_NBUF = 5


def _make_sc_gather(n_tokens):
    b_per_w = n_tokens // _NW
    n_chunks = b_per_w // _CH
    n_outer = n_chunks // _NBUF
    mesh = plsc.VectorSubcoreMesh(core_axis_name="c", subcore_axis_name="s")

    @functools.partial(
        pl.kernel,
        mesh=mesh,
        out_type=jax.ShapeDtypeStruct((n_tokens, EMBED), jnp.float32),
        scratch_types=[
            pltpu.VMEM((n_chunks, _CH), jnp.int32),
            pltpu.VMEM((_NBUF, _CH, EMBED), jnp.float32),
            pltpu.SemaphoreType.DMA((_NBUF,)),
            pltpu.SemaphoreType.DMA((_NBUF,)),
        ],
    )
    def gather_kernel(o_hbm, idx_hbm, out_hbm, idx_v, rows_v, gsem, wsem):
        wid = lax.axis_index("s") * 2 + lax.axis_index("c")
        base = wid * b_per_w
        pltpu.sync_copy(idx_hbm.at[wid], idx_v)

        def start_gather(j, b):
            pltpu.make_async_copy(
                o_hbm.at[idx_v.at[j]], rows_v.at[b], gsem.at[b]).start()

        def wait_gather(j, b):
            pltpu.make_async_copy(
                o_hbm.at[idx_v.at[j]], rows_v.at[b], gsem.at[b]).wait()

        def start_write(j, b):
            pltpu.make_async_copy(
                rows_v.at[b], out_hbm.at[pl.ds(base + j * _CH, _CH)],
                wsem.at[b]).start()

        def wait_write(b):
            pltpu.make_async_copy(
                rows_v.at[b], out_hbm.at[pl.ds(base, _CH)], wsem.at[b]).wait()

        for b in range(_NBUF - 1):
            start_gather(b, b)

        def outer(i, carry):
            for b in range(_NBUF):
                j = i * _NBUF + b
                nb = (b + _NBUF - 1) % _NBUF
                if b == 0:
                    @pl.when(i > 0)
                    def _():
                        wait_write(nb)
                    start_gather(j + _NBUF - 1, nb)
                else:
                    @pl.when(i < n_outer - 1)
                    def _():
                        wait_write(nb)
                        start_gather(j + _NBUF - 1, nb)
                wait_gather(j, b)
                start_write(j, b)
            return carry

        lax.fori_loop(0, n_outer, outer, 0)
        for b in range(_NBUF):
            wait_write(b)

    return gather_kernel


def kernel(input_ids, table, W_c, b_c, W_g, b_g):
    transformed = _transform_table(table, W_c, b_c, W_g, b_g)
    B, L = input_ids.shape
    flat_ids = input_ids.T.reshape(-1).astype(jnp.int32)
    n_tokens = flat_ids.shape[0]
    idx3 = flat_ids.reshape(_NW, n_tokens // (_NW * _CH), _CH)
    out = _make_sc_gather(n_tokens)(transformed, idx3)
    return out.reshape(L, B, EMBED).transpose(1, 0, 2)
